# Initial kernel scaffold; baseline (speedup 1.0000x reference)
import jax
import jax.numpy as jnp
from jax.experimental import pallas as pl


def kernel(x, edge_index, edge_weight, W_z, b_z, W_r, b_r, W_h, b_h, Wl_z, bl_z, Wl_r, bl_r, Wl_h, bl_h, W_out, b_out):
    raise NotImplementedError("write your pallas kernel here")


# speedup vs baseline: 24.5586x; 24.5586x over previous
import functools

import jax
import jax.numpy as jnp
from jax import lax
from jax.experimental import pallas as pl
from jax.experimental.pallas import tpu as pltpu
from jax.experimental.pallas import tpu_sc as plsc

N_NODES = 10000
N_PAD = 10240
N_EDGES = 320000
CH = 128
NW = 32
EPW = N_EDGES // NW
G = 80
NCH = EPW // G
EPS = N_EDGES // 16
NCS = EPS // G


@functools.cache
def _sc_kernels():
    mesh = plsc.VectorSubcoreMesh(core_axis_name="c", subcore_axis_name="s")

    @functools.partial(
        pl.kernel,
        mesh=mesh,
        out_type=jax.ShapeDtypeStruct((2 * N_PAD,), jnp.float32),
        scratch_types=[
            pltpu.VMEM((NCH, G), jnp.int32),
            pltpu.VMEM((NCH, G), jnp.float32),
            pltpu.VMEM((N_PAD // 16,), jnp.float32),
            pltpu.VMEM_SHARED((N_PAD,), jnp.float32),
        ],
    )
    def deg_kernel(dst_hbm, ew_hbm, out_hbm, dst_v, ew_v, zb_v, deg_sh):
        cid = lax.axis_index("c")
        sid = lax.axis_index("s")
        wid = sid * 2 + cid
        seg = N_PAD // 16

        def zloop(i, _):
            zb_v[pl.ds(i * 16, 16)] = jnp.zeros((16,), jnp.float32)
            return 0
        lax.fori_loop(0, seg // 16, zloop, 0)
        pltpu.sync_copy(zb_v, deg_sh.at[pl.ds(sid * seg, seg)])
        plsc.subcore_barrier()

        pltpu.sync_copy(dst_hbm.at[wid], dst_v)
        pltpu.sync_copy(ew_hbm.at[wid], ew_v)

        def body(c, _):
            pltpu.sync_copy(ew_v.at[c], deg_sh.at[dst_v.at[c]], add=True)
            return 0
        lax.fori_loop(0, NCH, body, 0)
        plsc.subcore_barrier()

        pltpu.sync_copy(deg_sh.at[pl.ds(sid * seg, seg)],
                        out_hbm.at[pl.ds(cid * N_PAD + sid * seg, seg)])

    @functools.partial(
        pl.kernel,
        mesh=mesh,
        compiler_params=pltpu.CompilerParams(use_tc_tiling_on_sc=False),
        out_type=jax.ShapeDtypeStruct((2, N_PAD, CH // 2), jnp.float32),
        scratch_types=[
            pltpu.VMEM((EPS,), jnp.int32),
            pltpu.VMEM((NCS, G), jnp.int32),
            pltpu.VMEM((EPS,), jnp.float32),
            pltpu.VMEM((G, CH // 2), jnp.float32),
            pltpu.VMEM((G, CH // 2), jnp.float32),
            pltpu.VMEM((128, CH // 2), jnp.float32),
            pltpu.VMEM_SHARED((N_PAD, CH // 2), jnp.float32),
            pltpu.SemaphoreType.DMA,
            pltpu.SemaphoreType.DMA,
        ],
    )
    def agg_kernel(src2_hbm, dst_hbm, ew_hbm, zv_hbm, out_hbm,
                   src_v, dst_v, ew_v, b0, b1, zb_v, agg_sh, sem0, sem1):
        cid = lax.axis_index("c")
        sid = lax.axis_index("s")
        rows = N_PAD // 16

        def zloop(i, _):
            zb_v[i // 4, pl.ds((i % 4) * 16, 16)] = jnp.zeros((16,), jnp.float32)
            return 0
        lax.fori_loop(0, 128 * 4, zloop, 0)
        for t in range(5):
            pltpu.sync_copy(zb_v, agg_sh.at[pl.ds(sid * rows + t * 128, 128)])
        plsc.subcore_barrier()

        pltpu.sync_copy(src2_hbm.at[sid], src_v)
        pltpu.sync_copy(dst_hbm.at[sid], dst_v)
        pltpu.sync_copy(ew_hbm.at[sid], ew_v)

        def adj(i, _):
            src_v[pl.ds(i * 16, 16)] = src_v[pl.ds(i * 16, 16)] + cid
            return 0
        lax.fori_loop(0, EPS // 16, adj, 0)

        def scale(buf, cc):
            base = cc * G

            def grouploop(q, _):
                wvec = ew_v[pl.ds(base + q * 16, 16)]
                for r in range(16):
                    w = lax.gather(
                        wvec, jnp.full((16, 1), r, jnp.int32),
                        lax.GatherDimensionNumbers(
                            offset_dims=(), collapsed_slice_dims=(0,),
                            start_index_map=(0,)),
                        (1,), mode=lax.GatherScatterMode.PROMISE_IN_BOUNDS)
                    j = q * 16 + r
                    for k in range(CH // 32):
                        buf[j, pl.ds(k * 16, 16)] = (
                            buf[j, pl.ds(k * 16, 16)] * w)
                return 0
            lax.fori_loop(0, G // 16, grouploop, 0)

        def gidx(c):
            return src_v.at[pl.ds(c * G, G)]

        pltpu.make_async_copy(zv_hbm.at[gidx(0)], b0, sem0).start()
        pltpu.make_async_copy(zv_hbm.at[gidx(1)], b1, sem1).start()

        def body(i, _):
            c0 = 2 * i
            pltpu.make_async_copy(zv_hbm.at[gidx(c0)], b0, sem0).wait()
            scale(b0, c0)
            pltpu.sync_copy(b0, agg_sh.at[dst_v.at[c0]], add=True)

            pltpu.make_async_copy(zv_hbm.at[gidx(c0 + 2)], b0, sem0).start()

            pltpu.make_async_copy(zv_hbm.at[gidx(c0 + 1)], b1, sem1).wait()
            scale(b1, c0 + 1)
            pltpu.sync_copy(b1, agg_sh.at[dst_v.at[c0 + 1]], add=True)

            @pl.when(i < NCS // 2 - 1)
            def _():
                pltpu.make_async_copy(
                    zv_hbm.at[gidx(c0 + 3)], b1, sem1).start()
            return 0
        lax.fori_loop(0, NCS // 2 - 1, body, 0)
        pltpu.make_async_copy(zv_hbm.at[gidx(NCS - 2)], b0, sem0).wait()
        scale(b0, NCS - 2)
        pltpu.sync_copy(b0, agg_sh.at[dst_v.at[NCS - 2]], add=True)
        pltpu.make_async_copy(zv_hbm.at[gidx(NCS - 1)], b1, sem1).wait()
        scale(b1, NCS - 1)
        pltpu.sync_copy(b1, agg_sh.at[dst_v.at[NCS - 1]], add=True)
        plsc.subcore_barrier()

        pltpu.sync_copy(agg_sh.at[pl.ds(sid * rows, rows)],
                        out_hbm.at[cid, pl.ds(sid * rows, rows)])

    return deg_kernel, agg_kernel


def _scale_body(degp_ref, x_ref, Wz_ref, bz_ref, Wlz_ref, blz_ref,
                Wh_ref, bh_ref, Wlh_ref, blh_ref,
                z_ref, dis_ref, WzF_ref, blzF_ref, WhF_ref, blhF_ref):
    deg = degp_ref[0, :N_NODES] + degp_ref[1, :N_NODES] + 1.0
    dis = jnp.where(deg > 0, lax.rsqrt(deg), 0.0)
    z_ref[...] = dis[:, None] * x_ref[...]
    dis_ref[...] = dis[:, None]
    WzF_ref[...] = jnp.dot(Wz_ref[...], Wlz_ref[...],
                           preferred_element_type=jnp.float32)
    blzF_ref[...] = jnp.dot(bz_ref[...], Wlz_ref[...],
                            preferred_element_type=jnp.float32) + blz_ref[...]
    WhF_ref[...] = jnp.dot(Wh_ref[...], Wlh_ref[...],
                           preferred_element_type=jnp.float32)
    blhF_ref[...] = jnp.dot(bh_ref[...], Wlh_ref[...],
                            preferred_element_type=jnp.float32) + blh_ref[...]


def _tc_scale(degp, x, Wz, bz, Wlz, blz, Wh, bh, Wlh, blh):
    return pl.pallas_call(
        _scale_body,
        out_shape=(
            jax.ShapeDtypeStruct((N_NODES, CH), jnp.float32),
            jax.ShapeDtypeStruct((N_NODES, 1), jnp.float32),
            jax.ShapeDtypeStruct((CH, CH), jnp.float32),
            jax.ShapeDtypeStruct((1, CH), jnp.float32),
            jax.ShapeDtypeStruct((CH, CH), jnp.float32),
            jax.ShapeDtypeStruct((1, CH), jnp.float32),
        ),
    )(degp, x, Wz, bz, Wlz, blz, Wh, bh, Wlh, blh)


_RB = 1000


def _dense_body(dis_ref, sp_ref, z_ref, Wz_ref, blz_ref, Wh_ref, blh_ref,
                Wo_ref, bo_ref, out_ref):
    s = jnp.concatenate([sp_ref[0], sp_ref[1]], axis=1)
    agg = dis_ref[...] * (s + z_ref[...])
    gz = jnp.dot(agg, Wz_ref[...], preferred_element_type=jnp.float32)
    zg = jax.nn.sigmoid(gz + blz_ref[...])
    gh = jnp.dot(agg, Wh_ref[...], preferred_element_type=jnp.float32)
    ht = jnp.tanh(gh + blh_ref[...])
    h = jax.nn.relu((1.0 - zg) * ht)
    out_ref[...] = (
        jnp.dot(h, Wo_ref[...], preferred_element_type=jnp.float32)
        + bo_ref[...])


def _tc_dense(dis, sp, z, Wz, blz, Wh, blh, Wo, bo):
    nblk = N_NODES // _RB
    return pl.pallas_call(
        _dense_body,
        grid=(nblk,),
        in_specs=[
            pl.BlockSpec((_RB, 1), lambda i: (i, 0)),
            pl.BlockSpec((2, _RB, CH // 2), lambda i: (0, i, 0)),
            pl.BlockSpec((_RB, CH), lambda i: (i, 0)),
            pl.BlockSpec((CH, CH), lambda i: (0, 0)),
            pl.BlockSpec((1, CH), lambda i: (0, 0)),
            pl.BlockSpec((CH, CH), lambda i: (0, 0)),
            pl.BlockSpec((1, CH), lambda i: (0, 0)),
            pl.BlockSpec((CH, 32), lambda i: (0, 0)),
            pl.BlockSpec((1, 32), lambda i: (0, 0)),
        ],
        out_specs=pl.BlockSpec((_RB, 32), lambda i: (i, 0)),
        out_shape=jax.ShapeDtypeStruct((N_NODES, 32), jnp.float32),
    )(dis, sp, z, Wz, blz, Wh, blh, Wo, bo)


def kernel(x, edge_index, edge_weight, W_z, b_z, W_r, b_r, W_h, b_h,
           Wl_z, bl_z, Wl_r, bl_r, Wl_h, bl_h, W_out, b_out):
    src_i = edge_index[0].astype(jnp.int32)
    dst_i = edge_index[1].astype(jnp.int32)
    ew_f = edge_weight.astype(jnp.float32)

    deg_kernel, agg_kernel = _sc_kernels()
    degp = deg_kernel(dst_i.reshape(NW, NCH, G),
                      ew_f.reshape(NW, NCH, G)).reshape(2, N_PAD)
    z, dis, WzF, blzF, WhF, blhF = _tc_scale(
        degp, x, W_z, b_z.reshape(1, CH), Wl_z[:CH], bl_z.reshape(1, CH),
        W_h, b_h.reshape(1, CH), Wl_h[:CH], bl_h.reshape(1, CH))
    sp = agg_kernel((src_i * 2).reshape(16, EPS),
                    dst_i.reshape(16, NCS, G),
                    ew_f.reshape(16, EPS),
                    z.reshape(2 * N_NODES, CH // 2))

    return _tc_dense(dis, sp, z, WzF, blzF, WhF, blhF,
                     W_out, b_out.reshape(1, 32))
